# baseline (speedup 1.0000x reference)
import jax
import jax.numpy as jnp
from jax import lax
from jax.experimental import pallas as pl
from jax.experimental.pallas import tpu as pltpu
from jax.experimental.pallas import tpu_sc as plsc

NUM_CORES = 2
NUM_SUBCORES = 16
NUM_WORKERS = NUM_CORES * NUM_SUBCORES

STREAM_ROWS = 128
STREAMS_PER_GROUP = 5
GROUP_ROWS = STREAM_ROWS * STREAMS_PER_GROUP


def _embed_gather(table, idx_flat, B, D):
    b_per_w = B // NUM_WORKERS
    num_groups = b_per_w // GROUP_ROWS
    assert b_per_w % GROUP_ROWS == 0 and num_groups % 2 == 0

    mesh = plsc.VectorSubcoreMesh(
        core_axis_name="c", subcore_axis_name="s",
        num_cores=NUM_CORES, num_subcores=NUM_SUBCORES)

    @pl.kernel(
        mesh=mesh,
        compiler_params=pltpu.CompilerParams(use_tc_tiling_on_sc=False),
        out_type=jax.ShapeDtypeStruct((B, D), jnp.float32),
        scratch_types=[
            pltpu.VMEM((b_per_w,), jnp.int32),
            pltpu.VMEM((GROUP_ROWS, D), jnp.float32),
            pltpu.VMEM((GROUP_ROWS, D), jnp.float32),
            pltpu.SemaphoreType.DMA,
            pltpu.SemaphoreType.DMA,
        ],
    )
    def k(table_hbm, idx_hbm, out_hbm, idx_v, buf0, buf1, sem0, sem1):
        wid = lax.axis_index("s") * NUM_CORES + lax.axis_index("c")
        base = wid * b_per_w
        pltpu.sync_copy(idx_hbm.at[pl.ds(base, b_per_w)], idx_v)

        def fire(g, buf, sem):
            off = g * GROUP_ROWS
            for s in range(STREAMS_PER_GROUP):
                pltpu.async_copy(
                    table_hbm.at[idx_v.at[pl.ds(off + s * STREAM_ROWS, STREAM_ROWS)]],
                    buf.at[pl.ds(s * STREAM_ROWS, STREAM_ROWS)],
                    sem)

        def drain_out(g, buf, sem):
            off = g * GROUP_ROWS
            for s in range(STREAMS_PER_GROUP):
                pltpu.make_async_copy(
                    table_hbm.at[idx_v.at[pl.ds(off + s * STREAM_ROWS, STREAM_ROWS)]],
                    buf.at[pl.ds(s * STREAM_ROWS, STREAM_ROWS)],
                    sem).wait()
            pltpu.sync_copy(buf, out_hbm.at[pl.ds(base + off, GROUP_ROWS)])

        fire(0, buf0, sem0)

        def body(j, carry):
            g0 = 2 * j
            fire(g0 + 1, buf1, sem1)
            drain_out(g0, buf0, sem0)

            @pl.when(g0 + 2 < num_groups)
            def _():
                fire(g0 + 2, buf0, sem0)

            drain_out(g0 + 1, buf1, sem1)
            return carry

        lax.fori_loop(0, num_groups // 2, body, 0)

    return k(table, idx_flat)


def kernel(inputs, embedding):
    batch, hist = inputs.shape
    num_emb, feat = embedding.shape
    idx_flat = inputs.reshape(-1).astype(jnp.int32)
    emb = jnp.asarray(embedding, jnp.float32)
    out = _embed_gather(emb, idx_flat, batch * hist, feat)
    return out.reshape(batch, hist, feat)
